# baseline (speedup 1.0000x reference)
import jax
import jax.numpy as jnp
from jax import lax
from jax.experimental import pallas as pl
from jax.experimental.pallas import tpu as pltpu

N = 32768
H = 256
B = 16
BLK = 8192
CH = 8192
NB = N // BLK


def _fused_kernel(batch_full_ref,
                  x_ref,
                  batch_ref,
                  nc_ref,
                  w1t_ref,
                  w2t_ref,
                  b12_ref,
                  q_ref,
                  qb_ref,
                  w3at_ref,
                  w3bt_ref,
                  b3_ref,
                  x_any_ref,
                  out_ref,
                  vn_ref,
                  a_ref,
                  sg_ref,
                  sem):
    i = pl.program_id(0)

    @pl.when(i == 0)
    def _prologue():
        sg_ref[...] = jnp.zeros_like(sg_ref)
        bf = batch_full_ref[...]
        cps = []
        for b in range(B):
            idx = jnp.sum((bf <= b).astype(jnp.int32)) - 1
            cp = pltpu.make_async_copy(
                x_any_ref.at[pl.ds(idx, 1), :],
                vn_ref.at[pl.ds(b, 1), :],
                sem)
            cp.start()
            cps.append(cp)
        for cp in cps:
            cp.wait()
        a_ref[...] = (
            jnp.dot(vn_ref[...], w1t_ref[...],
                    preferred_element_type=jnp.float32)
            + b12_ref[...])

    b_iota = lax.broadcasted_iota(jnp.int32, (B, CH), 0)
    acc = None
    for c in range(BLK // CH):
        sl = pl.ds(c * CH, CH)
        x_bf = x_ref[sl, :].astype(jnp.bfloat16)
        batch_row = batch_ref[0, :, sl]
        oh_t = (batch_row == b_iota).astype(jnp.float32)
        m = jnp.dot(x_bf, w2t_ref[...],
                    preferred_element_type=jnp.float32)
        a_tok = lax.dot_general(oh_t, a_ref[...],
                                (((0,), (0,)), ((), ())),
                                preferred_element_type=jnp.float32)
        z = m + a_tok
        gate = 1.0 / (1.0 + jnp.exp2(z * (-1.4426950408889634)))
        alpha = lax.dot_general(q_ref[...], gate,
                                (((1,), (1,)), ((), ())),
                                preferred_element_type=jnp.float32)
        scale = (alpha + qb_ref[...]) * nc_ref[0, :, sl]
        d = jnp.dot((oh_t * scale).astype(jnp.bfloat16), x_bf,
                    preferred_element_type=jnp.float32)
        acc = d if acc is None else acc + d
    sg_ref[...] += acc

    @pl.when(i == NB - 1)
    def _epilogue():
        out_ref[...] = (
            jnp.dot(vn_ref[...], w3at_ref[...],
                    preferred_element_type=jnp.float32)
            + jnp.dot(sg_ref[...], w3bt_ref[...],
                      preferred_element_type=jnp.float32)
            + b3_ref[...])


@jax.jit
def kernel(node_embedding, batch, num_count, W1_w, W1_b, W2_w, W2_b,
           q_w, q_b, W3_w, W3_b):
    batch_full = batch.reshape(8, N // 8)
    batch_blk = batch.reshape(NB, 1, BLK)
    nc_blk = num_count.reshape(NB, 1, BLK)
    b12 = (W1_b + W2_b).reshape(1, H)
    qb = q_b.reshape(1, 1)
    w3at = W3_w[:, :H].T
    w3bt = W3_w[:, H:].T
    b3 = W3_b.reshape(1, H)

    const2 = lambda i: (0, 0)
    out = pl.pallas_call(
        _fused_kernel,
        grid=(NB,),
        in_specs=[
            pl.BlockSpec((8, N // 8), const2),
            pl.BlockSpec((BLK, H), lambda i: (i, 0)),
            pl.BlockSpec((1, 1, BLK), lambda i: (i, 0, 0)),
            pl.BlockSpec((1, 1, BLK), lambda i: (i, 0, 0)),
            pl.BlockSpec((H, H), const2),
            pl.BlockSpec((H, H), const2),
            pl.BlockSpec((1, H), const2),
            pl.BlockSpec((1, H), const2),
            pl.BlockSpec((1, 1), const2),
            pl.BlockSpec((H, H), const2),
            pl.BlockSpec((H, H), const2),
            pl.BlockSpec((1, H), const2),
            pl.BlockSpec(memory_space=pl.ANY),
        ],
        out_specs=pl.BlockSpec((B, H), const2),
        out_shape=jax.ShapeDtypeStruct((B, H), jnp.float32),
        scratch_shapes=[
            pltpu.VMEM((B, H), jnp.float32),
            pltpu.VMEM((B, H), jnp.float32),
            pltpu.VMEM((B, H), jnp.float32),
            pltpu.SemaphoreType.DMA,
        ],
        compiler_params=pltpu.CompilerParams(
            dimension_semantics=("arbitrary",)),
    )(batch_full, node_embedding, batch_blk, nc_blk,
      W1_w.T, W2_w.T.astype(jnp.bfloat16), b12, q_w, qb, w3at, w3bt, b3,
      node_embedding)
    return out


# speedup vs baseline: 1.0330x; 1.0330x over previous
import jax
import jax.numpy as jnp
from jax import lax
from jax.experimental import pallas as pl
from jax.experimental.pallas import tpu as pltpu

N = 32768
H = 256
B = 16
BLK = 4096
CH = 4096
NB = N // BLK


def _fused_kernel(batch_full_ref,
                  x_ref,
                  batch_ref,
                  nc_ref,
                  w1t_ref,
                  w2t_ref,
                  b12_ref,
                  q_ref,
                  qb_ref,
                  w3at_ref,
                  w3bt_ref,
                  b3_ref,
                  x_any_ref,
                  out_ref,
                  vn_ref,
                  a_ref,
                  sg_ref,
                  sem):
    i = pl.program_id(0)

    @pl.when(i == 0)
    def _prologue():
        sg_ref[...] = jnp.zeros_like(sg_ref)
        bf = batch_full_ref[...]
        cps = []
        for b in range(B):
            idx = jnp.sum((bf <= b).astype(jnp.int32)) - 1
            cp = pltpu.make_async_copy(
                x_any_ref.at[pl.ds(idx, 1), :],
                vn_ref.at[pl.ds(b, 1), :],
                sem)
            cp.start()
            cps.append(cp)
        for cp in cps:
            cp.wait()
        a_ref[...] = (
            jnp.dot(vn_ref[...], w1t_ref[...],
                    preferred_element_type=jnp.float32)
            + b12_ref[...])

    b_iota = lax.broadcasted_iota(jnp.int32, (B, CH), 0)
    acc = None
    for c in range(BLK // CH):
        sl = pl.ds(c * CH, CH)
        x_bf = x_ref[sl, :].astype(jnp.bfloat16)
        batch_row = batch_ref[0, :, sl]
        oh_t = (batch_row == b_iota).astype(jnp.float32)
        m = jnp.dot(x_bf, w2t_ref[...],
                    preferred_element_type=jnp.float32)
        a_tok = lax.dot_general(oh_t, a_ref[...],
                                (((0,), (0,)), ((), ())),
                                preferred_element_type=jnp.float32)
        z = m + a_tok
        gate = 1.0 / (1.0 + jnp.exp2(z * (-1.4426950408889634)))
        alpha = lax.dot_general(q_ref[...], gate,
                                (((1,), (1,)), ((), ())),
                                preferred_element_type=jnp.float32)
        scale = (alpha + qb_ref[...]) * nc_ref[0, :, sl]
        d = jnp.dot((oh_t * scale).astype(jnp.bfloat16), x_bf,
                    preferred_element_type=jnp.float32)
        acc = d if acc is None else acc + d
    sg_ref[...] += acc

    @pl.when(i == NB - 1)
    def _epilogue():
        out_ref[...] = (
            jnp.dot(vn_ref[...], w3at_ref[...],
                    preferred_element_type=jnp.float32)
            + jnp.dot(sg_ref[...], w3bt_ref[...],
                      preferred_element_type=jnp.float32)
            + b3_ref[...])


@jax.jit
def kernel(node_embedding, batch, num_count, W1_w, W1_b, W2_w, W2_b,
           q_w, q_b, W3_w, W3_b):
    batch_full = batch.reshape(8, N // 8)
    batch_blk = batch.reshape(NB, 1, BLK)
    nc_blk = num_count.reshape(NB, 1, BLK)
    b12 = (W1_b + W2_b).reshape(1, H)
    qb = q_b.reshape(1, 1)
    w3at = W3_w[:, :H].T
    w3bt = W3_w[:, H:].T
    b3 = W3_b.reshape(1, H)

    const2 = lambda i: (0, 0)
    out = pl.pallas_call(
        _fused_kernel,
        grid=(NB,),
        in_specs=[
            pl.BlockSpec((8, N // 8), const2),
            pl.BlockSpec((BLK, H), lambda i: (i, 0)),
            pl.BlockSpec((1, 1, BLK), lambda i: (i, 0, 0)),
            pl.BlockSpec((1, 1, BLK), lambda i: (i, 0, 0)),
            pl.BlockSpec((H, H), const2),
            pl.BlockSpec((H, H), const2),
            pl.BlockSpec((1, H), const2),
            pl.BlockSpec((1, H), const2),
            pl.BlockSpec((1, 1), const2),
            pl.BlockSpec((H, H), const2),
            pl.BlockSpec((H, H), const2),
            pl.BlockSpec((1, H), const2),
            pl.BlockSpec(memory_space=pl.ANY),
        ],
        out_specs=pl.BlockSpec((B, H), const2),
        out_shape=jax.ShapeDtypeStruct((B, H), jnp.float32),
        scratch_shapes=[
            pltpu.VMEM((B, H), jnp.float32),
            pltpu.VMEM((B, H), jnp.float32),
            pltpu.VMEM((B, H), jnp.float32),
            pltpu.SemaphoreType.DMA,
        ],
        compiler_params=pltpu.CompilerParams(
            dimension_semantics=("arbitrary",)),
    )(batch_full, node_embedding, batch_blk, nc_blk,
      W1_w.T, W2_w.T.astype(jnp.bfloat16), b12, q_w, qb, w3at, w3bt, b3,
      node_embedding)
    return out


# speedup vs baseline: 1.3272x; 1.2849x over previous
import jax
import jax.numpy as jnp
from jax import lax
from jax.experimental import pallas as pl
from jax.experimental.pallas import tpu as pltpu

N = 32768
H = 256
B = 16
BLK = 4096
CH = 4096
NB = N // BLK


def _fused_kernel(batch_full_ref,
                  x_ref,
                  batch_ref,
                  nc_ref,
                  w1_ref,
                  w2_ref,
                  b1_ref,
                  b2_ref,
                  q_ref,
                  qb_ref,
                  w3_ref,
                  b3_ref,
                  x_any_ref,
                  out_ref,
                  vn_ref,
                  a_ref,
                  sg_ref,
                  w2bf_ref,
                  sem):
    i = pl.program_id(0)

    @pl.when(i == 0)
    def _prologue():
        sg_ref[...] = jnp.zeros_like(sg_ref)
        bf = batch_full_ref[...]
        cps = []
        for b in range(B):
            idx = jnp.sum((bf <= b).astype(jnp.int32)) - 1
            cp = pltpu.make_async_copy(
                x_any_ref.at[pl.ds(idx, 1), :],
                vn_ref.at[pl.ds(b, 1), :],
                sem)
            cp.start()
            cps.append(cp)
        for cp in cps:
            cp.wait()
        a_ref[...] = (
            lax.dot_general(vn_ref[...], w1_ref[...],
                            (((1,), (1,)), ((), ())),
                            preferred_element_type=jnp.float32)
            + b1_ref[...] + b2_ref[...])
        w2bf_ref[...] = w2_ref[...].astype(jnp.bfloat16)

    b_iota = lax.broadcasted_iota(jnp.int32, (B, CH), 0)
    acc = None
    for c in range(BLK // CH):
        sl = pl.ds(c * CH, CH)
        x_bf = x_ref[sl, :].astype(jnp.bfloat16)
        batch_row = batch_ref[0, :, sl]
        oh_t = (batch_row == b_iota).astype(jnp.float32)
        m = lax.dot_general(x_bf, w2bf_ref[...],
                            (((1,), (1,)), ((), ())),
                            preferred_element_type=jnp.float32)
        a_tok = lax.dot_general(oh_t, a_ref[...],
                                (((0,), (0,)), ((), ())),
                                preferred_element_type=jnp.float32)
        z = m + a_tok
        gate = 1.0 / (1.0 + jnp.exp2(z * (-1.4426950408889634)))
        alpha = lax.dot_general(q_ref[...], gate,
                                (((1,), (1,)), ((), ())),
                                preferred_element_type=jnp.float32)
        scale = (alpha + qb_ref[...]) * nc_ref[0, :, sl]
        d = jnp.dot((oh_t * scale).astype(jnp.bfloat16), x_bf,
                    preferred_element_type=jnp.float32)
        acc = d if acc is None else acc + d
    sg_ref[...] += acc

    @pl.when(i == NB - 1)
    def _epilogue():
        out_ref[...] = (
            lax.dot_general(vn_ref[...], w3_ref[:, :H],
                            (((1,), (1,)), ((), ())),
                            preferred_element_type=jnp.float32)
            + lax.dot_general(sg_ref[...], w3_ref[:, H:],
                              (((1,), (1,)), ((), ())),
                              preferred_element_type=jnp.float32)
            + b3_ref[...])


@jax.jit
def kernel(node_embedding, batch, num_count, W1_w, W1_b, W2_w, W2_b,
           q_w, q_b, W3_w, W3_b):
    batch_full = batch.reshape(8, N // 8)
    batch_blk = batch.reshape(NB, 1, BLK)
    nc_blk = num_count.reshape(NB, 1, BLK)

    const2 = lambda i: (0, 0)
    out = pl.pallas_call(
        _fused_kernel,
        grid=(NB,),
        in_specs=[
            pl.BlockSpec((8, N // 8), const2),
            pl.BlockSpec((BLK, H), lambda i: (i, 0)),
            pl.BlockSpec((1, 1, BLK), lambda i: (i, 0, 0)),
            pl.BlockSpec((1, 1, BLK), lambda i: (i, 0, 0)),
            pl.BlockSpec((H, H), const2),
            pl.BlockSpec((H, H), const2),
            pl.BlockSpec((1, H), const2),
            pl.BlockSpec((1, H), const2),
            pl.BlockSpec((1, H), const2),
            pl.BlockSpec((1, 1), const2),
            pl.BlockSpec((H, 2 * H), const2),
            pl.BlockSpec((1, H), const2),
            pl.BlockSpec(memory_space=pl.ANY),
        ],
        out_specs=pl.BlockSpec((B, H), const2),
        out_shape=jax.ShapeDtypeStruct((B, H), jnp.float32),
        scratch_shapes=[
            pltpu.VMEM((B, H), jnp.float32),
            pltpu.VMEM((B, H), jnp.float32),
            pltpu.VMEM((B, H), jnp.float32),
            pltpu.VMEM((H, H), jnp.bfloat16),
            pltpu.SemaphoreType.DMA,
        ],
        compiler_params=pltpu.CompilerParams(
            dimension_semantics=("arbitrary",)),
    )(batch_full, node_embedding, batch_blk, nc_blk,
      W1_w, W2_w, W1_b.reshape(1, H), W2_b.reshape(1, H), q_w,
      q_b.reshape(1, 1), W3_w, W3_b.reshape(1, H),
      node_embedding)
    return out
